# baseline (speedup 1.0000x reference)
import jax
import jax.numpy as jnp
from jax.experimental import pallas as pl
from jax.experimental.pallas import tpu as pltpu

_IN = 4096
_OUT = 4096
_K = _IN // 2
_BLK = 512


def _body(x_ref, w_ref, b_ref, o_ref, xm_ref):
    @pl.when(pl.program_id(0) == 0)
    def _select():
        xv = x_ref[...]
        x8 = jnp.reshape(xv, (8, _IN // 8))
        bits = jax.lax.bitcast_convert_type(jnp.abs(x8), jnp.int32)

        kf = jnp.float32(_K)

        thr = jnp.int32(0)
        for shift, nvals in ((28, 7), (24, 15), (20, 15), (16, 15),
                             (12, 15), (8, 15), (4, 15), (0, 15)):
            step = jnp.float32(0.0)
            for v in range(1, nvals + 1):
                cand = thr | jnp.int32(v << shift)
                cnt = jnp.sum(jnp.where(bits >= cand, 1.0, 0.0))
                step = step + jnp.where(cnt >= kf, 1.0, 0.0)
            thr = thr | (step.astype(jnp.int32) << shift)

        gt = bits > thr
        mf = kf - jnp.sum(jnp.where(gt, 1.0, 0.0))
        eq = bits == thr
        eqf = jnp.where(eq, 1.0, 0.0)
        r = jax.lax.broadcasted_iota(jnp.int32, (8, _IN // 8), 0)
        c = jax.lax.broadcasted_iota(jnp.int32, (8, _IN // 8), 1)
        idx = r * (_IN // 8) + c

        bound = jnp.int32(0)
        for shift, nvals in ((12, 1), (8, 15), (4, 15), (0, 15)):
            step = jnp.float32(0.0)
            for v in range(1, nvals + 1):
                cand2 = bound | jnp.int32(v << shift)
                cnt2 = jnp.sum(jnp.where(idx < cand2, eqf, 0.0))
                step = step + jnp.where(cnt2 <= mf, 1.0, 0.0)
            bound = bound | (step.astype(jnp.int32) << shift)

        mask = gt | (eq & (idx < bound))
        xm8 = jnp.where(mask, x8, 0.0)
        xm_ref[...] = jnp.reshape(xm8, (1, _IN))

    xm = xm_ref[...]
    acc = jax.lax.dot_general(
        w_ref[...], xm, (((1,), (1,)), ((), ())),
        preferred_element_type=jnp.float32)
    accT = jax.lax.transpose(acc, (1, 0))
    o_ref[...] = accT + b_ref[...]


def kernel(x, W, b):
    x2 = x.reshape(1, _IN)
    b2 = b.reshape(1, _OUT)
    out = pl.pallas_call(
        _body,
        grid=(_OUT // _BLK,),
        in_specs=[
            pl.BlockSpec((1, _IN), lambda i: (0, 0)),
            pl.BlockSpec((_BLK, _IN), lambda i: (i, 0)),
            pl.BlockSpec((1, _BLK), lambda i: (0, i)),
        ],
        out_specs=pl.BlockSpec((1, _BLK), lambda i: (0, i)),
        out_shape=jax.ShapeDtypeStruct((1, _OUT), jnp.float32),
        scratch_shapes=[pltpu.VMEM((1, _IN), jnp.float32)],
        compiler_params=pltpu.CompilerParams(
            dimension_semantics=("arbitrary",)),
    )(x2, W, b2)
    return out.reshape(1, 1, _OUT)
